# baseline (speedup 1.0000x reference)
import functools

import numpy as np
import jax
import jax.numpy as jnp
from jax import lax
from jax.experimental import pallas as pl
from jax.experimental.pallas import tpu as pltpu
from jax.experimental.pallas import tpu_sc as plsc

_NUM_SPHERICAL = 7
_NUM_RADIAL = 6
_CUTOFF = 5.0
_NK = _NUM_SPHERICAL * _NUM_RADIAL
_NKP = 48
_DP = 128

_EB = 2560
_TB = 1024

_PREF = np.sqrt((2 * np.arange(_NUM_SPHERICAL) + 1) / (4.0 * np.pi)).astype(
    np.float32
)


def _rbf_body(d_ref, z_ref, n_ref, out_ref):
    d = d_ref[...].reshape(1, _EB) / np.float32(_CUTOFF)
    zs = z_ref[...]

    xs = d * zs
    sin_ = jnp.sin(xs)
    cos_ = jnp.cos(xs)

    grp = lax.broadcasted_iota(jnp.int32, (_NKP, _EB), 0) // _NUM_RADIAL

    j0 = sin_ / xs
    j1 = sin_ / (xs * xs) - cos_ / xs
    res = jnp.where(grp == 0, j0, j1)
    jm, jc = j0, j1
    for i in range(1, _NUM_SPHERICAL - 1):
        jn = np.float32(2 * i + 1) / xs * jc - jm
        res = jnp.where(grp == i + 1, jn, res)
        jm, jc = jc, jn
    rbf = res * n_ref[...]
    out_ref[...] = jnp.concatenate(
        [rbf.T, jnp.zeros((_EB, _DP - _NKP), jnp.float32)], axis=1
    )


def _rbf_table(dist, bessel_zeros, bessel_norm):
    E = dist.shape[0]
    nblk = E // _EB
    zcol = jnp.concatenate(
        [bessel_zeros.reshape(_NK, 1), jnp.ones((_NKP - _NK, 1), jnp.float32)]
    )
    ncol = jnp.concatenate(
        [bessel_norm.reshape(_NK, 1), jnp.zeros((_NKP - _NK, 1), jnp.float32)]
    )
    d3 = dist.reshape(nblk, 1, _EB)
    return pl.pallas_call(
        _rbf_body,
        grid=(nblk,),
        in_specs=[
            pl.BlockSpec((1, 1, _EB), lambda i: (i, 0, 0)),
            pl.BlockSpec((_NKP, 1), lambda i: (0, 0)),
            pl.BlockSpec((_NKP, 1), lambda i: (0, 0)),
        ],
        out_specs=pl.BlockSpec((_EB, _DP), lambda i: (i, 0)),
        out_shape=jax.ShapeDtypeStruct((E, _DP), jnp.float32),
    )(d3, zcol, ncol)


def _sc_gather_rows(table, idx_kj):
    T = idx_kj.shape[0]
    info = plsc.get_sparse_core_info()
    nw = info.num_cores * info.num_subcores
    tpw = T // nw
    ch = 80
    nch = tpw // ch

    mesh = plsc.VectorSubcoreMesh(core_axis_name="c", subcore_axis_name="s")

    @functools.partial(
        pl.kernel,
        mesh=mesh,
        out_type=jax.ShapeDtypeStruct((T, _DP), jnp.float32),
        scratch_types=[
            pltpu.VMEM((tpw,), jnp.int32),
            pltpu.VMEM((2 * ch, _DP), jnp.float32),
            pltpu.SemaphoreType.DMA,
            pltpu.SemaphoreType.DMA,
        ],
    )
    def gather_kernel(tab_hbm, idx_hbm, out_hbm, idx_v, rows_v, gsem, osem):
        wid = lax.axis_index("s") * info.num_cores + lax.axis_index("c")
        base = wid * tpw
        pltpu.sync_copy(idx_hbm.at[pl.ds(base, tpw)], idx_v)

        def gcopy(g):
            slot = lax.rem(g, 2) * ch
            return pltpu.make_async_copy(
                tab_hbm.at[idx_v.at[pl.ds(g * ch, ch)]],
                rows_v.at[pl.ds(slot, ch)],
                gsem,
            )

        def ocopy(g):
            slot = lax.rem(g, 2) * ch
            return pltpu.make_async_copy(
                rows_v.at[pl.ds(slot, ch)],
                out_hbm.at[pl.ds(base + g * ch, ch)],
                osem,
            )

        gcopy(0).start()

        def body(g, carry):
            gcopy(g).wait()

            @pl.when(g >= 1)
            def _():
                ocopy(g - 1).wait()

            @pl.when(g + 1 < nch)
            def _():
                gcopy(g + 1).start()

            ocopy(g).start()
            return carry

        lax.fori_loop(0, nch, body, 0)
        ocopy(nch - 1).wait()

    return gather_kernel(table, idx_kj)


def _mul_body(g_ref, a_ref, sel_ref, out_ref):
    z = jnp.cos(a_ref[...].reshape(1, _TB))
    ps = [jnp.ones_like(z), z]
    for l in range(2, _NUM_SPHERICAL):
        ps.append(
            (np.float32(2 * l - 1) * z * ps[-1] - np.float32(l - 1) * ps[-2])
            / np.float32(l)
        )
    ps.append(jnp.zeros_like(z))
    p8 = jnp.concatenate(ps, axis=0)
    cbf = jax.lax.dot_general(
        p8.T,
        sel_ref[...],
        (((1,), (0,)), ((), ())),
        preferred_element_type=jnp.float32,
    )
    out_ref[...] = (g_ref[...] * cbf)[:, :_NK]


def _mul_stage(g, angle):
    T = angle.shape[0]
    nblk = T // _TB
    sel = np.zeros((8, _DP), np.float32)
    for s in range(_NUM_SPHERICAL):
        sel[s, s * _NUM_RADIAL : (s + 1) * _NUM_RADIAL] = _PREF[s]
    a3 = angle.reshape(nblk, 1, _TB)
    return pl.pallas_call(
        _mul_body,
        grid=(nblk,),
        in_specs=[
            pl.BlockSpec((_TB, _DP), lambda i: (i, 0)),
            pl.BlockSpec((1, 1, _TB), lambda i: (i, 0, 0)),
            pl.BlockSpec((8, _DP), lambda i: (0, 0)),
        ],
        out_specs=pl.BlockSpec((_TB, _NK), lambda i: (i, 0)),
        out_shape=jax.ShapeDtypeStruct((T, _NK), jnp.float32),
    )(g, a3, jnp.asarray(sel))


def kernel(dist, angle, idx_kj, bessel_zeros, bessel_norm):
    table = _rbf_table(dist, bessel_zeros, bessel_norm)
    g = _sc_gather_rows(table, idx_kj)
    return _mul_stage(g, angle)


# speedup vs baseline: 1.6845x; 1.6845x over previous
import functools

import numpy as np
import jax
import jax.numpy as jnp
from jax import lax
from jax.experimental import pallas as pl
from jax.experimental.pallas import tpu as pltpu
from jax.experimental.pallas import tpu_sc as plsc

_NUM_SPHERICAL = 7
_NUM_RADIAL = 6
_CUTOFF = 5.0
_NK = _NUM_SPHERICAL * _NUM_RADIAL
_NKP = 48

_TB = 2560

_PREF = np.sqrt((2 * np.arange(_NUM_SPHERICAL) + 1) / (4.0 * np.pi)).astype(
    np.float32
)


def _sc_gather(dist, idx_kj):
    T = idx_kj.shape[0]
    info = plsc.get_sparse_core_info()
    nw = info.num_cores * info.num_subcores
    tpw = T // nw
    ch = 80
    nch = tpw // ch
    depth = 8

    mesh = plsc.VectorSubcoreMesh(core_axis_name="c", subcore_axis_name="s")

    @functools.partial(
        pl.kernel,
        mesh=mesh,
        out_type=jax.ShapeDtypeStruct((T,), jnp.float32),
        scratch_types=[
            pltpu.VMEM((tpw,), jnp.int32),
            pltpu.VMEM((tpw,), jnp.float32),
            pltpu.SemaphoreType.DMA,
        ],
    )
    def gather_kernel(dist_hbm, idx_hbm, out_hbm, idx_v, d_v, sem):
        wid = lax.axis_index("s") * info.num_cores + lax.axis_index("c")
        base = wid * tpw
        pltpu.sync_copy(idx_hbm.at[pl.ds(base, tpw)], idx_v)

        def chunk_copy(g):
            return pltpu.make_async_copy(
                dist_hbm.at[idx_v.at[pl.ds(g * ch, ch)]],
                d_v.at[pl.ds(g * ch, ch)],
                sem,
            )

        def fire(g, carry):
            chunk_copy(g).start()

            @pl.when(g >= depth)
            def _():
                chunk_copy(g - depth).wait()

            return carry

        lax.fori_loop(0, nch, fire, 0)

        def drain(g, carry):
            chunk_copy(nch - depth + g).wait()
            return carry

        lax.fori_loop(0, depth, drain, 0)
        pltpu.sync_copy(d_v, out_hbm.at[pl.ds(base, tpw)])

    return gather_kernel(dist, idx_kj)


def _tc_body(d_ref, a_ref, z_ref, n_ref, p_ref, out_ref):
    d = d_ref[...].reshape(1, _TB) / np.float32(_CUTOFF)
    zs = z_ref[...]

    xs = d * zs
    sin_ = jnp.sin(xs)
    cos_ = jnp.cos(xs)

    j0 = sin_ / xs
    j1 = sin_ / (xs * xs) - cos_ / xs
    js = [j0, j1]
    jm, jc = j0, j1
    for i in range(1, _NUM_SPHERICAL - 1):
        jn = np.float32(2 * i + 1) / xs * jc - jm
        js.append(jn)
        jm, jc = jc, jn
    nr = _NUM_RADIAL
    res = jnp.concatenate(
        [js[s][s * nr : (s + 1) * nr] for s in range(_NUM_SPHERICAL)]
        + [js[0][_NK:_NKP]],
        axis=0,
    )
    rbf = res * n_ref[...]

    z = jnp.cos(a_ref[...].reshape(1, _TB))
    ps = [jnp.ones_like(z), z]
    pm, pc = ps[0], z
    for l in range(2, _NUM_SPHERICAL):
        pn = (np.float32(2 * l - 1) * z * pc - np.float32(l - 1) * pm) / np.float32(l)
        ps.append(pn)
        pm, pc = pc, pn
    cb = jnp.concatenate(
        [jnp.broadcast_to(ps[s], (nr, _TB)) for s in range(_NUM_SPHERICAL)]
        + [jnp.zeros((_NKP - _NK, _TB), jnp.float32)],
        axis=0,
    )
    cbf = cb * p_ref[...]

    out = rbf * cbf
    out_ref[...] = out.T[:, :_NK]


def _tc_math(d_g, angle, bessel_zeros, bessel_norm):
    T = d_g.shape[0]
    nblk = T // _TB

    zcol = jnp.concatenate(
        [
            bessel_zeros.reshape(_NK, 1),
            jnp.ones((_NKP - _NK, 1), jnp.float32),
        ]
    )
    ncol = jnp.concatenate(
        [
            bessel_norm.reshape(_NK, 1),
            jnp.zeros((_NKP - _NK, 1), jnp.float32),
        ]
    )
    pcol = jnp.concatenate(
        [
            jnp.asarray(np.repeat(_PREF, _NUM_RADIAL).reshape(_NK, 1)),
            jnp.zeros((_NKP - _NK, 1), jnp.float32),
        ]
    )
    d3 = d_g.reshape(nblk, 1, _TB)
    a3 = angle.reshape(nblk, 1, _TB)

    return pl.pallas_call(
        _tc_body,
        grid=(nblk,),
        in_specs=[
            pl.BlockSpec((1, 1, _TB), lambda i: (i, 0, 0)),
            pl.BlockSpec((1, 1, _TB), lambda i: (i, 0, 0)),
            pl.BlockSpec((_NKP, 1), lambda i: (0, 0)),
            pl.BlockSpec((_NKP, 1), lambda i: (0, 0)),
            pl.BlockSpec((_NKP, 1), lambda i: (0, 0)),
        ],
        out_specs=pl.BlockSpec((_TB, _NK), lambda i: (i, 0)),
        out_shape=jax.ShapeDtypeStruct((T, _NK), jnp.float32),
    )(d3, a3, zcol, ncol, pcol)


def kernel(dist, angle, idx_kj, bessel_zeros, bessel_norm):
    d_g = _sc_gather(dist, idx_kj)
    return _tc_math(d_g, angle, bessel_zeros, bessel_norm)
